# baseline (speedup 1.0000x reference)
import functools

import jax
import jax.numpy as jnp
import numpy as np
from jax import lax
from jax.experimental import pallas as pl
from jax.experimental.pallas import tpu as pltpu
from jax.experimental.pallas import tpu_sc as plsc

N = 2048
HID = 1024
NE = 7
INTER = 1365
T = 128
G = 4992
NT = G // T
NC, NS = 2, 16
NW = NC * NS
BPW = G // NW
GCH = 16
GNB = 6
TPW = N // NW

_F32 = jnp.float32
_I32 = jnp.int32

_L128 = np.tril(np.ones((128, 128), np.float32), k=-1)
_U128 = np.triu(np.ones((128, 128), np.float32), k=1)


def _router_perm_body(h_ref, wr_ref, br_ref, gb_ref, l_ref, u_ref,
                      pos1_ref, pos2_ref, w1_ref, w2_ref, te_ref):
    h = h_ref[...]
    gate = jnp.dot(h, wr_ref[...], preferred_element_type=_F32) + br_ref[...]
    bg = gate + gb_ref[...]
    lane = lax.broadcasted_iota(_I32, (N, 128), 1)
    big = jnp.int32(1 << 30)
    m1 = jnp.max(bg, axis=1, keepdims=True)
    a1 = jnp.min(jnp.where(bg == m1, lane, big), axis=1, keepdims=True)
    bg2 = jnp.where(lane == a1, -jnp.inf, bg)
    m2 = jnp.max(bg2, axis=1, keepdims=True)
    a2 = jnp.min(jnp.where(bg2 == m2, lane, big), axis=1, keepdims=True)
    g1 = jnp.sum(jnp.where(lane == a1, gate, 0.0), axis=1, keepdims=True)
    g2 = jnp.sum(jnp.where(lane == a2, gate, 0.0), axis=1, keepdims=True)
    mx = jnp.maximum(g1, g2)
    e1 = jnp.exp(g1 - mx)
    e2 = jnp.exp(g2 - mx)
    s = e1 + e2
    w1_ref[...] = e1 / s
    w2_ref[...] = e2 / s
    c1 = (lane == a1).astype(_F32)
    c2 = (lane == a2).astype(_F32)
    cnt = c1 + c2
    ltri = l_ref[...]
    parts = []
    run = jnp.zeros((1, 128), _F32)
    for b in range(N // 128):
        cb = cnt[b * 128:(b + 1) * 128, :]
        parts.append(jnp.dot(ltri, cb, preferred_element_type=_F32) + run)
        run = run + jnp.sum(cb, axis=0, keepdims=True)
    pfx = jnp.concatenate(parts, axis=0)
    counts = run.astype(_I32)
    ntp = ((counts + (T - 1)) // T) * T
    ntp_f = ntp.astype(_F32)
    start = jnp.dot(ntp_f, u_ref[...], preferred_element_type=_F32)
    spos = start + pfx
    pos1 = jnp.sum(jnp.where(lane == a1, spos, 0.0), axis=1, keepdims=True)
    pos2 = jnp.sum(jnp.where(lane == a2, spos, 0.0), axis=1, keepdims=True)
    pos1_ref[...] = pos1.astype(_I32)
    pos2_ref[...] = pos2.astype(_I32)
    lrow = lane[:1, :]
    tlane = lax.broadcasted_iota(_I32, (8, 128), 1)
    til_f = (tlane * T).astype(_F32)
    te = jnp.zeros((8, 128), _I32)
    for e in range(NE):
        se = jnp.sum(jnp.where(lrow == e, start, 0.0))
        ne_ = jnp.sum(jnp.where(lrow == e, ntp_f, 0.0))
        te = te + (til_f >= se + ne_).astype(_I32)
    te = jnp.minimum(te, NE - 1)
    used = (jnp.sum(ntp_f) / T).astype(_I32)
    te_ref[...] = jnp.where(tlane == NT, used, te)


def _router_perm(h, wr_pad, br_pad, gb_pad):
    out_shapes = [
        jax.ShapeDtypeStruct((N, 1), _I32),
        jax.ShapeDtypeStruct((N, 1), _I32),
        jax.ShapeDtypeStruct((N, 1), _F32),
        jax.ShapeDtypeStruct((N, 1), _F32),
        jax.ShapeDtypeStruct((8, 128), _I32),
    ]
    return pl.pallas_call(_router_perm_body, out_shape=out_shapes)(
        h, wr_pad, br_pad, gb_pad, jnp.asarray(_L128), jnp.asarray(_U128))


@functools.lru_cache(maxsize=None)
def _sc_mesh():
    return plsc.VectorSubcoreMesh(core_axis_name="c", subcore_axis_name="s",
                                  num_cores=NC, num_subcores=NS)


@functools.lru_cache(maxsize=None)
def _scatter_build_kernel():
    return pl.kernel(
        _scatter_build_body,
        out_type=[jax.ShapeDtypeStruct((G,), _I32),
                  jax.ShapeDtypeStruct((G,), _F32)],
        mesh=_sc_mesh(),
        compiler_params=pltpu.CompilerParams(needs_layout_passes=False),
        scratch_types=[pltpu.VMEM((N,), _I32), pltpu.VMEM((N,), _I32),
                       pltpu.VMEM((N,), _F32), pltpu.VMEM((N,), _F32),
                       pltpu.VMEM((G,), _I32), pltpu.VMEM((G,), _F32)],
    )


def _scatter_build_body(pos1_hbm, pos2_hbm, w1_hbm, w2_hbm, gidx_hbm, wg_hbm,
                        p1v, p2v, w1v, w2v, gidx_v, wg_v):
    wid = lax.axis_index("s") * NC + lax.axis_index("c")

    @pl.when(wid == 0)
    def _():
        pltpu.sync_copy(pos1_hbm, p1v)
        pltpu.sync_copy(pos2_hbm, p2v)
        pltpu.sync_copy(w1_hbm, w1v)
        pltpu.sync_copy(w2_hbm, w2v)

        def zero_body(i, _):
            gidx_v[pl.ds(i * 16, 16)] = jnp.zeros((16,), _I32)
            wg_v[pl.ds(i * 16, 16)] = jnp.zeros((16,), _F32)
            return 0
        lax.fori_loop(0, G // 16, zero_body, 0)

        def body(t, _):
            base = t * 16
            p1 = p1v[pl.ds(base, 16)]
            p2 = p2v[pl.ds(base, 16)]
            wa = w1v[pl.ds(base, 16)]
            wb = w2v[pl.ds(base, 16)]
            toks = base + lax.iota(_I32, 16)
            plsc.store_scatter(gidx_v, [p1], toks)
            plsc.store_scatter(gidx_v, [p2], toks)
            plsc.store_scatter(wg_v, [p1], wa)
            plsc.store_scatter(wg_v, [p2], wb)
            return 0
        lax.fori_loop(0, N // 16, body, 0)
        pltpu.sync_copy(gidx_v, gidx_hbm)
        pltpu.sync_copy(wg_v, wg_hbm)


@functools.lru_cache(maxsize=None)
def _scatter_rows_kernel():
    return pl.kernel(
        _scatter_rows_body,
        out_type=jax.ShapeDtypeStruct((G, HID), _F32),
        mesh=_sc_mesh(),
        compiler_params=pltpu.CompilerParams(needs_layout_passes=False),
        scratch_types=[pltpu.VMEM((TPW, HID), _F32),
                       pltpu.VMEM((NW, TPW), _I32),
                       pltpu.VMEM((NW, TPW), _I32),
                       pltpu.SemaphoreType.DMA, pltpu.SemaphoreType.DMA],
    )


def _scatter_rows_body(h_hbm, pos1_hbm, pos2_hbm, hg_hbm, hv, p1v, p2v,
                       sem1, sem2):
    wid = lax.axis_index("s") * NC + lax.axis_index("c")
    pltpu.sync_copy(h_hbm.at[pl.ds(wid * TPW, TPW)], hv)
    pltpu.sync_copy(pos1_hbm, p1v)
    pltpu.sync_copy(pos2_hbm, p2v)
    d1 = pltpu.async_copy(hv, hg_hbm.at[p1v.at[wid]], sem1)
    d2 = pltpu.async_copy(hv, hg_hbm.at[p2v.at[wid]], sem2)
    d1.wait()
    d2.wait()


def _ffn_body(te_ref, hg_ref, wg_ref, wu_ref, wd_ref, bg_ref, bu_ref, bd_ref,
              wrow_ref, out_ref):
    @pl.when(pl.program_id(0) < te_ref[NT])
    def _():
        hgt = hg_ref[...]
        a = jnp.dot(hgt, wg_ref[0], preferred_element_type=_F32) + bg_ref[0]
        b = jnp.dot(hgt, wu_ref[0], preferred_element_type=_F32) + bu_ref[0]
        c = a * jax.nn.sigmoid(a) * b
        o = jnp.dot(c, wd_ref[0], preferred_element_type=_F32) + bd_ref[0]
        out_ref[...] = o * wrow_ref[...]


def _ffn_body_bf16(te_ref, hg_ref, wg_ref, wu_ref, wd_ref, bg_ref, bu_ref,
                   bd_ref, wrow_ref, out_ref):
    @pl.when(pl.program_id(0) < te_ref[NT])
    def _():
        hgt = hg_ref[...].astype(jnp.bfloat16)
        a = jnp.dot(hgt, wg_ref[0].astype(jnp.bfloat16),
                    preferred_element_type=_F32) + bg_ref[0]
        b = jnp.dot(hgt, wu_ref[0].astype(jnp.bfloat16),
                    preferred_element_type=_F32) + bu_ref[0]
        c = (a * jax.nn.sigmoid(a) * b).astype(jnp.bfloat16)
        o = jnp.dot(c, wd_ref[0].astype(jnp.bfloat16),
                    preferred_element_type=_F32) + bd_ref[0]
        out_ref[...] = o * wrow_ref[...]


def _ffn_grouped(te, hg, wg, wu, wd, bg, bu, bd, wrow, lowp):
    grid_spec = pltpu.PrefetchScalarGridSpec(
        num_scalar_prefetch=1,
        grid=(NT,),
        in_specs=[
            pl.BlockSpec((T, HID), lambda i, te: (i, 0)),
            pl.BlockSpec((1, HID, INTER), lambda i, te: (te[i], 0, 0)),
            pl.BlockSpec((1, HID, INTER), lambda i, te: (te[i], 0, 0)),
            pl.BlockSpec((1, INTER, HID), lambda i, te: (te[i], 0, 0)),
            pl.BlockSpec((1, 1, INTER), lambda i, te: (te[i], 0, 0)),
            pl.BlockSpec((1, 1, INTER), lambda i, te: (te[i], 0, 0)),
            pl.BlockSpec((1, 1, HID), lambda i, te: (te[i], 0, 0)),
            pl.BlockSpec((T, 1), lambda i, te: (i, 0)),
        ],
        out_specs=pl.BlockSpec((T, HID), lambda i, te: (i, 0)),
    )
    return pl.pallas_call(
        _ffn_body_bf16 if lowp else _ffn_body, grid_spec=grid_spec,
        out_shape=jax.ShapeDtypeStruct((G, HID), _F32),
    )(te, hg, wg, wu, wd, bg, bu, bd, wrow)


def _shared_body(h_ref, wup_ref, bup_ref, wsw_ref, bsw_ref, wdn_ref, bdn_ref,
                 out_ref):
    hh = h_ref[...]
    s1 = jnp.dot(hh, wup_ref[...], preferred_element_type=_F32) + bup_ref[...]
    z = jnp.dot(s1, wsw_ref[...], preferred_element_type=_F32) + bsw_ref[...]
    s2 = z * jax.nn.sigmoid(z) * s1
    out_ref[...] = (jnp.dot(s2, wdn_ref[...], preferred_element_type=_F32)
                    + bdn_ref[...])


def _shared_body_bf16(h_ref, wup_ref, bup_ref, wsw_ref, bsw_ref, wdn_ref,
                      bdn_ref, out_ref):
    hh = h_ref[...].astype(jnp.bfloat16)
    s1 = jnp.dot(hh, wup_ref[...].astype(jnp.bfloat16),
                 preferred_element_type=_F32) + bup_ref[...]
    z = jnp.dot(s1.astype(jnp.bfloat16), wsw_ref[...].astype(jnp.bfloat16),
                preferred_element_type=_F32) + bsw_ref[...]
    s2 = (z * jax.nn.sigmoid(z) * s1).astype(jnp.bfloat16)
    out_ref[...] = (jnp.dot(s2, wdn_ref[...].astype(jnp.bfloat16),
                            preferred_element_type=_F32) + bdn_ref[...])


def _shared_expert(h, wup, bup, wsw, bsw, wdn, bdn, lowp):
    grid_spec = pl.GridSpec(
        grid=(N // T,),
        in_specs=[
            pl.BlockSpec((T, HID), lambda i: (i, 0)),
            pl.BlockSpec((HID, INTER), lambda i: (0, 0)),
            pl.BlockSpec((1, INTER), lambda i: (0, 0)),
            pl.BlockSpec((INTER, INTER), lambda i: (0, 0)),
            pl.BlockSpec((1, INTER), lambda i: (0, 0)),
            pl.BlockSpec((INTER, HID), lambda i: (0, 0)),
            pl.BlockSpec((1, HID), lambda i: (0, 0)),
        ],
        out_specs=pl.BlockSpec((T, HID), lambda i: (i, 0)),
    )
    return pl.pallas_call(
        _shared_body_bf16 if lowp else _shared_body, grid_spec=grid_spec,
        out_shape=jax.ShapeDtypeStruct((N, HID), _F32),
    )(h, wup, bup, wsw, bsw, wdn, bdn)


@functools.lru_cache(maxsize=None)
def _combine_kernel():
    return pl.kernel(
        _combine_body,
        out_type=jax.ShapeDtypeStruct((N, HID), _F32),
        mesh=_sc_mesh(),
        compiler_params=pltpu.CompilerParams(needs_layout_passes=False),
        scratch_types=[pltpu.VMEM((TPW,), _I32), pltpu.VMEM((TPW,), _I32),
                       pltpu.VMEM((16, HID), _F32), pltpu.VMEM((16, HID), _F32),
                       pltpu.VMEM((16, HID), _F32), pltpu.VMEM((16, HID), _F32),
                       pltpu.SemaphoreType.DMA, pltpu.SemaphoreType.DMA],
    )


def _combine_body(pos1_hbm, pos2_hbm, outg_hbm, sh_hbm, hnew_hbm,
                  p1v, p2v, r0, r1, shv, ob, sem, sem2):
    wid = lax.axis_index("s") * NC + lax.axis_index("c")
    pltpu.sync_copy(pos1_hbm.at[pl.ds(wid * TPW, TPW)], p1v)
    pltpu.sync_copy(pos2_hbm.at[pl.ds(wid * TPW, TPW)], p2v)

    def chunk_body(cc, _):
        t0 = wid * TPW + cc * 16
        d0 = pltpu.async_copy(outg_hbm.at[p1v.at[pl.ds(cc * 16, 16)]], r0,
                              sem)
        d1 = pltpu.async_copy(outg_hbm.at[p2v.at[pl.ds(cc * 16, 16)]], r1,
                              sem2)
        pltpu.sync_copy(sh_hbm.at[pl.ds(t0, 16)], shv)
        d0.wait()
        d1.wait()
        for i in range(16):
            def col_body(cj, _):
                c = cj * 64
                for u in range(4):
                    sl = pl.ds(c + u * 16, 16)
                    ob[i, sl] = r0[i, sl] + r1[i, sl] + shv[i, sl]
                return 0
            lax.fori_loop(0, HID // 64, col_body, 0)
        pltpu.sync_copy(ob, hnew_hbm.at[pl.ds(t0, 16)])
        return 0

    lax.fori_loop(0, TPW // 16, chunk_body, 0)


def _one_chain(h, params, j):
    wr = params["router_W"][j]
    wr_pad = jnp.pad(wr, ((0, 0), (0, 128 - NE)))
    br_pad = jnp.pad(params["router_b"][j], (0, 128 - NE)).reshape(1, 128)
    gb_pad = jnp.pad(params["gate_bias"][j], (0, 128 - NE),
                     constant_values=-1e30).reshape(1, 128)

    pos1b, pos2b, w1b, w2b, teb = _router_perm(h, wr_pad, br_pad, gb_pad)
    pos1, pos2 = pos1b.reshape(N), pos2b.reshape(N)
    w1, w2 = w1b.reshape(N), w2b.reshape(N)
    gidx, wg_flat = _scatter_build_kernel()(pos1, pos2, w1, w2)
    del gidx
    hg = _scatter_rows_kernel()(h, pos1b.reshape(NW, TPW),
                                pos2b.reshape(NW, TPW))
    te = teb[0, :NT + 1]
    lowp = (j == 1)
    out_g = _ffn_grouped(
        te, hg,
        params["exp_Wg"], params["exp_Wu"], params["exp_Wd"],
        params["exp_bg"].reshape(NE, 1, INTER),
        params["exp_bu"].reshape(NE, 1, INTER),
        params["exp_bd"].reshape(NE, 1, HID),
        wg_flat.reshape(G, 1), lowp)
    sh = _shared_expert(
        h, params["share_up_W"], params["share_up_b"].reshape(1, INTER),
        params["swiglu_W"], params["swiglu_b"].reshape(1, INTER),
        params["share_down_W"], params["share_down_b"].reshape(1, HID), lowp)
    return _combine_kernel()(pos1, pos2, out_g, sh)


def kernel(x, params):
    shape = x.shape
    h = x.reshape(-1, shape[-1])
    for j in range(2):
        h = _one_chain(h, params, j)
    return h.reshape(shape)


# speedup vs baseline: 1.0246x; 1.0246x over previous
import functools

import jax
import jax.numpy as jnp
import numpy as np
from jax import lax
from jax.experimental import pallas as pl
from jax.experimental.pallas import tpu as pltpu
from jax.experimental.pallas import tpu_sc as plsc

N = 2048
HID = 1024
NE = 7
INTER = 1365
T = 128
G = 4992
NT = G // T
NC, NS = 2, 16
NW = NC * NS
BPW = G // NW
GCH = 16
GNB = 6
TPW = N // NW

_F32 = jnp.float32
_I32 = jnp.int32

_L128 = np.tril(np.ones((128, 128), np.float32), k=-1)
_U128 = np.triu(np.ones((128, 128), np.float32), k=1)
_I128 = np.eye(128, dtype=np.float32)
_L16 = np.tril(np.ones((16, 16), np.float32), k=-1)


def _cols_to_rows(cols, ident):
    rows = []
    for r in range(N // 128):
        blk = cols[r * 128:(r + 1) * 128, :]
        rows.append(lax.dot_general(blk, ident, (((0,), (0,)), ((), ())),
                                    precision=lax.Precision.HIGHEST,
                                    preferred_element_type=_F32))
    return jnp.concatenate(rows, axis=0)


def _router_perm_body(h_ref, wr_ref, br_ref, gb_ref, l_ref, u_ref, i_ref,
                      l16_ref, pw_ref, te_ref):
    ident = i_ref[...]
    h = h_ref[...]
    gate = jnp.dot(h, wr_ref[...], preferred_element_type=_F32) + br_ref[...]
    bg = gate + gb_ref[...]
    lane = lax.broadcasted_iota(_I32, (N, 128), 1)
    big = jnp.int32(1 << 30)
    m1 = jnp.max(bg, axis=1, keepdims=True)
    a1 = jnp.min(jnp.where(bg == m1, lane, big), axis=1, keepdims=True)
    bg2 = jnp.where(lane == a1, -jnp.inf, bg)
    m2 = jnp.max(bg2, axis=1, keepdims=True)
    a2 = jnp.min(jnp.where(bg2 == m2, lane, big), axis=1, keepdims=True)
    g1 = jnp.sum(jnp.where(lane == a1, gate, 0.0), axis=1, keepdims=True)
    g2 = jnp.sum(jnp.where(lane == a2, gate, 0.0), axis=1, keepdims=True)
    mx = jnp.maximum(g1, g2)
    e1 = jnp.exp(g1 - mx)
    e2 = jnp.exp(g2 - mx)
    s = e1 + e2
    c1 = (lane == a1).astype(_F32)
    c2 = (lane == a2).astype(_F32)
    cnt = c1 + c2
    ltri = l_ref[...]
    bsums = [jnp.sum(cnt[b * 128:(b + 1) * 128, :], axis=0, keepdims=True)
             for b in range(N // 128)]
    bsum = jnp.concatenate(bsums, axis=0)
    boff = jnp.dot(l16_ref[...], bsum, preferred_element_type=_F32)
    parts = []
    for b in range(N // 128):
        cb = cnt[b * 128:(b + 1) * 128, :]
        parts.append(jnp.dot(ltri, cb, preferred_element_type=_F32)
                     + boff[b:b + 1, :])
    pfx = jnp.concatenate(parts, axis=0)
    counts = (boff[15:16, :] + bsum[15:16, :]).astype(_I32)
    ntp = ((counts + (T - 1)) // T) * T
    ntp_f = ntp.astype(_F32)
    start = jnp.dot(ntp_f, u_ref[...], preferred_element_type=_F32)
    spos = start + pfx
    pos1 = jnp.sum(jnp.where(lane == a1, spos, 0.0), axis=1, keepdims=True)
    pos2 = jnp.sum(jnp.where(lane == a2, spos, 0.0), axis=1, keepdims=True)
    pw = jnp.concatenate([pos1, pos2, e1 / s, e2 / s], axis=1)
    pw_ref[...] = _cols_to_rows(pw, ident)
    lrow = lane[:1, :]
    tlane = lax.broadcasted_iota(_I32, (8, 128), 1)
    til_f = (tlane * T).astype(_F32)
    te = jnp.zeros((8, 128), _I32)
    for e in range(NE):
        se = jnp.sum(jnp.where(lrow == e, start, 0.0))
        ne_ = jnp.sum(jnp.where(lrow == e, ntp_f, 0.0))
        te = te + (til_f >= se + ne_).astype(_I32)
    te = jnp.minimum(te, NE - 1)
    used = (jnp.sum(ntp_f) / T).astype(_I32)
    te_ref[...] = jnp.where(tlane == NT, used, te)


def _router_perm(h, wr_pad, br_pad, gb_pad):
    out_shapes = [
        jax.ShapeDtypeStruct((N // 128 * 4, 128), _F32),
        jax.ShapeDtypeStruct((8, 128), _I32),
    ]
    return pl.pallas_call(_router_perm_body, out_shape=out_shapes)(
        h, wr_pad, br_pad, gb_pad, jnp.asarray(_L128), jnp.asarray(_U128),
        jnp.asarray(_I128), jnp.asarray(_L16))


@functools.lru_cache(maxsize=None)
def _sc_mesh():
    return plsc.VectorSubcoreMesh(core_axis_name="c", subcore_axis_name="s",
                                  num_cores=NC, num_subcores=NS)


@functools.lru_cache(maxsize=None)
def _scatter_build_kernel():
    return pl.kernel(
        _scatter_build_body,
        out_type=[jax.ShapeDtypeStruct((G,), _I32),
                  jax.ShapeDtypeStruct((G,), _F32)],
        mesh=_sc_mesh(),
        compiler_params=pltpu.CompilerParams(needs_layout_passes=False),
        scratch_types=[pltpu.VMEM((N,), _I32), pltpu.VMEM((N,), _I32),
                       pltpu.VMEM((N,), _F32), pltpu.VMEM((N,), _F32),
                       pltpu.VMEM((G,), _I32), pltpu.VMEM((G,), _F32)],
    )


def _scatter_build_body(pos1_hbm, pos2_hbm, w1_hbm, w2_hbm, gidx_hbm, wg_hbm,
                        p1v, p2v, w1v, w2v, gidx_v, wg_v):
    wid = lax.axis_index("s") * NC + lax.axis_index("c")

    @pl.when(wid == 0)
    def _():
        pltpu.sync_copy(pos1_hbm, p1v)
        pltpu.sync_copy(pos2_hbm, p2v)
        pltpu.sync_copy(w1_hbm, w1v)
        pltpu.sync_copy(w2_hbm, w2v)

        def zero_body(i, _):
            gidx_v[pl.ds(i * 16, 16)] = jnp.zeros((16,), _I32)
            wg_v[pl.ds(i * 16, 16)] = jnp.zeros((16,), _F32)
            return 0
        lax.fori_loop(0, G // 16, zero_body, 0)

        def body(t, _):
            base = t * 16
            p1 = p1v[pl.ds(base, 16)]
            p2 = p2v[pl.ds(base, 16)]
            wa = w1v[pl.ds(base, 16)]
            wb = w2v[pl.ds(base, 16)]
            toks = base + lax.iota(_I32, 16)
            plsc.store_scatter(gidx_v, [p1], toks)
            plsc.store_scatter(gidx_v, [p2], toks)
            plsc.store_scatter(wg_v, [p1], wa)
            plsc.store_scatter(wg_v, [p2], wb)
            return 0
        lax.fori_loop(0, N // 16, body, 0)
        pltpu.sync_copy(gidx_v, gidx_hbm)
        pltpu.sync_copy(wg_v, wg_hbm)


@functools.lru_cache(maxsize=None)
def _scatter_rows_kernel():
    return pl.kernel(
        _scatter_rows_body,
        out_type=jax.ShapeDtypeStruct((G, HID), _F32),
        mesh=_sc_mesh(),
        compiler_params=pltpu.CompilerParams(needs_layout_passes=False),
        scratch_types=[pltpu.VMEM((TPW, HID), _F32),
                       pltpu.VMEM((NW, TPW), _I32),
                       pltpu.VMEM((NW, TPW), _I32),
                       pltpu.SemaphoreType.DMA, pltpu.SemaphoreType.DMA],
    )


def _scatter_rows_body(h_hbm, pos1_hbm, pos2_hbm, hg_hbm, hv, p1v, p2v,
                       sem1, sem2):
    wid = lax.axis_index("s") * NC + lax.axis_index("c")
    pltpu.sync_copy(h_hbm.at[pl.ds(wid * TPW, TPW)], hv)
    pltpu.sync_copy(pos1_hbm, p1v)
    pltpu.sync_copy(pos2_hbm, p2v)
    d1 = pltpu.async_copy(hv, hg_hbm.at[p1v.at[wid]], sem1)
    d2 = pltpu.async_copy(hv, hg_hbm.at[p2v.at[wid]], sem2)
    d1.wait()
    d2.wait()


def _ffn_body(te_ref, hg_ref, wg_ref, wu_ref, wd_ref, bg_ref, bu_ref, bd_ref,
              wrow_ref, out_ref):
    @pl.when(pl.program_id(0) < te_ref[NT])
    def _():
        hgt = hg_ref[...]
        a = jnp.dot(hgt, wg_ref[0], preferred_element_type=_F32) + bg_ref[0]
        b = jnp.dot(hgt, wu_ref[0], preferred_element_type=_F32) + bu_ref[0]
        c = a * jax.nn.sigmoid(a) * b
        o = jnp.dot(c, wd_ref[0], preferred_element_type=_F32) + bd_ref[0]
        out_ref[...] = o * wrow_ref[...]


def _ffn_body_bf16(te_ref, hg_ref, wg_ref, wu_ref, wd_ref, bg_ref, bu_ref,
                   bd_ref, wrow_ref, out_ref):
    @pl.when(pl.program_id(0) < te_ref[NT])
    def _():
        hgt = hg_ref[...].astype(jnp.bfloat16)
        a = jnp.dot(hgt, wg_ref[0].astype(jnp.bfloat16),
                    preferred_element_type=_F32) + bg_ref[0]
        b = jnp.dot(hgt, wu_ref[0].astype(jnp.bfloat16),
                    preferred_element_type=_F32) + bu_ref[0]
        c = (a * jax.nn.sigmoid(a) * b).astype(jnp.bfloat16)
        o = jnp.dot(c, wd_ref[0].astype(jnp.bfloat16),
                    preferred_element_type=_F32) + bd_ref[0]
        out_ref[...] = o * wrow_ref[...]


def _ffn_grouped(te, hg, wg, wu, wd, bg, bu, bd, wrow, lowp):
    grid_spec = pltpu.PrefetchScalarGridSpec(
        num_scalar_prefetch=1,
        grid=(NT,),
        in_specs=[
            pl.BlockSpec((T, HID), lambda i, te: (i, 0)),
            pl.BlockSpec((1, HID, INTER), lambda i, te: (te[i], 0, 0)),
            pl.BlockSpec((1, HID, INTER), lambda i, te: (te[i], 0, 0)),
            pl.BlockSpec((1, INTER, HID), lambda i, te: (te[i], 0, 0)),
            pl.BlockSpec((1, 1, INTER), lambda i, te: (te[i], 0, 0)),
            pl.BlockSpec((1, 1, INTER), lambda i, te: (te[i], 0, 0)),
            pl.BlockSpec((1, 1, HID), lambda i, te: (te[i], 0, 0)),
            pl.BlockSpec((T, 1), lambda i, te: (i, 0)),
        ],
        out_specs=pl.BlockSpec((T, HID), lambda i, te: (i, 0)),
    )
    return pl.pallas_call(
        _ffn_body_bf16 if lowp else _ffn_body, grid_spec=grid_spec,
        out_shape=jax.ShapeDtypeStruct((G, HID), _F32),
    )(te, hg, wg, wu, wd, bg, bu, bd, wrow)


def _shared_body(h_ref, wup_ref, bup_ref, wsw_ref, bsw_ref, wdn_ref, bdn_ref,
                 out_ref):
    hh = h_ref[...]
    s1 = jnp.dot(hh, wup_ref[...], preferred_element_type=_F32) + bup_ref[...]
    z = jnp.dot(s1, wsw_ref[...], preferred_element_type=_F32) + bsw_ref[...]
    s2 = z * jax.nn.sigmoid(z) * s1
    out_ref[...] = (jnp.dot(s2, wdn_ref[...], preferred_element_type=_F32)
                    + bdn_ref[...])


def _shared_body_bf16(h_ref, wup_ref, bup_ref, wsw_ref, bsw_ref, wdn_ref,
                      bdn_ref, out_ref):
    hh = h_ref[...].astype(jnp.bfloat16)
    s1 = jnp.dot(hh, wup_ref[...].astype(jnp.bfloat16),
                 preferred_element_type=_F32) + bup_ref[...]
    z = jnp.dot(s1.astype(jnp.bfloat16), wsw_ref[...].astype(jnp.bfloat16),
                preferred_element_type=_F32) + bsw_ref[...]
    s2 = (z * jax.nn.sigmoid(z) * s1).astype(jnp.bfloat16)
    out_ref[...] = (jnp.dot(s2, wdn_ref[...].astype(jnp.bfloat16),
                            preferred_element_type=_F32) + bdn_ref[...])


def _shared_expert(h, wup, bup, wsw, bsw, wdn, bdn, lowp):
    grid_spec = pl.GridSpec(
        grid=(N // T,),
        in_specs=[
            pl.BlockSpec((T, HID), lambda i: (i, 0)),
            pl.BlockSpec((HID, INTER), lambda i: (0, 0)),
            pl.BlockSpec((1, INTER), lambda i: (0, 0)),
            pl.BlockSpec((INTER, INTER), lambda i: (0, 0)),
            pl.BlockSpec((1, INTER), lambda i: (0, 0)),
            pl.BlockSpec((INTER, HID), lambda i: (0, 0)),
            pl.BlockSpec((1, HID), lambda i: (0, 0)),
        ],
        out_specs=pl.BlockSpec((T, HID), lambda i: (i, 0)),
    )
    return pl.pallas_call(
        _shared_body_bf16 if lowp else _shared_body, grid_spec=grid_spec,
        out_shape=jax.ShapeDtypeStruct((N, HID), _F32),
    )(h, wup, bup, wsw, bsw, wdn, bdn)


@functools.lru_cache(maxsize=None)
def _combine_kernel():
    return pl.kernel(
        _combine_body,
        out_type=jax.ShapeDtypeStruct((N, HID), _F32),
        mesh=_sc_mesh(),
        compiler_params=pltpu.CompilerParams(needs_layout_passes=False),
        scratch_types=[pltpu.VMEM((TPW,), _I32), pltpu.VMEM((TPW,), _I32),
                       pltpu.VMEM((16, HID), _F32), pltpu.VMEM((16, HID), _F32),
                       pltpu.VMEM((16, HID), _F32), pltpu.VMEM((16, HID), _F32),
                       pltpu.SemaphoreType.DMA, pltpu.SemaphoreType.DMA],
    )


def _combine_body(pos1_hbm, pos2_hbm, outg_hbm, sh_hbm, hnew_hbm,
                  p1v, p2v, r0, r1, shv, ob, sem, sem2):
    wid = lax.axis_index("s") * NC + lax.axis_index("c")
    pltpu.sync_copy(pos1_hbm.at[pl.ds(wid * TPW, TPW)], p1v)
    pltpu.sync_copy(pos2_hbm.at[pl.ds(wid * TPW, TPW)], p2v)

    def chunk_body(cc, _):
        t0 = wid * TPW + cc * 16
        d0 = pltpu.async_copy(outg_hbm.at[p1v.at[pl.ds(cc * 16, 16)]], r0,
                              sem)
        d1 = pltpu.async_copy(outg_hbm.at[p2v.at[pl.ds(cc * 16, 16)]], r1,
                              sem2)
        pltpu.sync_copy(sh_hbm.at[pl.ds(t0, 16)], shv)
        d0.wait()
        d1.wait()
        for i in range(16):
            def col_body(cj, _):
                c = cj * 64
                for u in range(4):
                    sl = pl.ds(c + u * 16, 16)
                    ob[i, sl] = r0[i, sl] + r1[i, sl] + shv[i, sl]
                return 0
            lax.fori_loop(0, HID // 64, col_body, 0)
        pltpu.sync_copy(ob, hnew_hbm.at[pl.ds(t0, 16)])
        return 0

    lax.fori_loop(0, TPW // 16, chunk_body, 0)


def _one_chain(h, params, j):
    wr = params["router_W"][j]
    wr_pad = jnp.pad(wr, ((0, 0), (0, 128 - NE)))
    br_pad = jnp.pad(params["router_b"][j], (0, 128 - NE)).reshape(1, 128)
    gb_pad = jnp.pad(params["gate_bias"][j], (0, 128 - NE),
                     constant_values=-1e30).reshape(1, 128)

    pwb, teb = _router_perm(h, wr_pad, br_pad, gb_pad)
    pw4 = pwb.reshape(N // 128, 4, 128)
    pos1b = pw4[:, 0, :].astype(_I32)
    pos2b = pw4[:, 1, :].astype(_I32)
    pos1, pos2 = pos1b.reshape(N), pos2b.reshape(N)
    w1 = pw4[:, 2, :].reshape(N)
    w2 = pw4[:, 3, :].reshape(N)
    gidx, wg_flat = _scatter_build_kernel()(pos1, pos2, w1, w2)
    del gidx
    hg = _scatter_rows_kernel()(h, pos1b.reshape(NW, TPW),
                                pos2b.reshape(NW, TPW))
    te = teb[0, :NT + 1]
    lowp = (j == 1)
    out_g = _ffn_grouped(
        te, hg,
        params["exp_Wg"], params["exp_Wu"], params["exp_Wd"],
        params["exp_bg"].reshape(NE, 1, INTER),
        params["exp_bu"].reshape(NE, 1, INTER),
        params["exp_bd"].reshape(NE, 1, HID),
        wg_flat.reshape(G, 1), lowp)
    sh = _shared_expert(
        h, params["share_up_W"], params["share_up_b"].reshape(1, INTER),
        params["swiglu_W"], params["swiglu_b"].reshape(1, INTER),
        params["share_down_W"], params["share_down_b"].reshape(1, HID), lowp)
    return _combine_kernel()(pos1, pos2, out_g, sh)


def kernel(x, params):
    shape = x.shape
    h = x.reshape(-1, shape[-1])
    for j in range(2):
        h = _one_chain(h, params, j)
    return h.reshape(shape)
